# baseline (speedup 1.0000x reference)
import jax
import jax.numpy as jnp
from jax.experimental import pallas as pl
from jax.experimental.pallas import tpu as pltpu

_V = 27
_VP = 32
_S = 4
_CFG = [(3, 64, 1, False), (64, 64, 1, True), (64, 64, 1, True), (64, 64, 1, True),
        (64, 128, 2, True), (128, 128, 1, True), (128, 128, 1, True),
        (128, 256, 2, True), (256, 256, 1, True), (256, 256, 1, True)]

_NC = 60


def _r(v):
    return v.reshape(1, -1)


def _prep_layer(lp):
    g = lp['gcn']
    pa = jnp.pad(g['PA'], ((0, 0), (0, _VP - _V), (0, _VP - _V)))
    p = {
        'PA': pa,
        'wa': list(g['wa']), 'ba': [_r(b) for b in g['ba']],
        'wb': list(g['wb']), 'bb': [_r(b) for b in g['bb']],
        'wd': list(g['wd']), 'bd': [_r(b) for b in g['bd']],
        'bn_g': _r(g['bn_g']), 'bn_b': _r(g['bn_b']),
    }
    if 'down_w' in g:
        p['down_w'] = g['down_w']
        p['down_b'] = _r(g['down_b'])
        p['down_g'] = _r(g['down_g'])
        p['down_bb'] = _r(g['down_bb'])
    t = lp['tcn']
    p['tw'] = jnp.transpose(t['w'][:, :, :, 0], (2, 1, 0))
    p['tb'] = _r(t['b'])
    p['tg'] = _r(t['g'])
    p['tbb'] = _r(t['bb'])
    if 'res' in lp:
        r = lp['res']
        p['rw'] = r['w'][:, :, 0, 0].T
        p['rb'] = _r(r['b'])
        p['rg'] = _r(r['g'])
        p['rbb'] = _r(r['bb'])
    return p


def _mm(a, b):
    return jnp.dot(a, b, preferred_element_type=jnp.float32)


def _dg(a, b, dims):
    return jax.lax.dot_general(a, b, dims, preferred_element_type=jnp.float32)


def _layer_body(X, x4, p, o_ref, sa, sb, sxa, sp,
                cin, cout, T, stride, residual, last):
    G = _S * _VP
    R = T * G
    inter = cout // 4
    Xw = [jnp.concatenate([x4[t, s] for t in range(T)], axis=1)
          for s in range(_S)]
    sa4 = sa.reshape(T, _S, _VP, inter)
    sb4 = sb.reshape(T, _S, _VP, inter)
    sxa4 = sxa.reshape(T, _S, _VP, cin)
    vmask = jax.lax.broadcasted_iota(jnp.int32, (_VP, _VP), 0) < _V
    y = None
    for i in range(3):
        sa[...] = _mm(X, p['wa'][i]) + p['ba'][i]
        sb[...] = _mm(X, p['wb'][i]) + p['bb'][i]
        for s in range(_S):
            c1 = jnp.concatenate([sa4[t, s] for t in range(T)], axis=1)
            c2 = jnp.concatenate([sb4[t, s] for t in range(T)], axis=1)
            logits = _dg(c1, c2, (((1,), (1,)), ((), ()))) / float(inter * T)
            logits = jnp.where(vmask, logits, -1e30)
            mx = jnp.max(logits, axis=0, keepdims=True)
            e = jnp.exp(logits - mx)
            att = e / jnp.sum(e, axis=0, keepdims=True)
            A1 = att + p['PA'][i]
            xa_wide = _dg(A1, Xw[s], (((0,), (0,)), ((), ())))
            for t in range(T):
                sxa4[t, s] = xa_wide[:, t * cin:(t + 1) * cin]
        contrib = _mm(sxa[...], p['wd'][i]) + p['bd'][i]
        y = contrib if y is None else y + contrib
    y = p['bn_g'] * y + p['bn_b']
    if 'down_w' in p:
        r = _mm(X, p['down_w']) + p['down_b']
        r = p['down_g'] * r + p['down_bb']
    else:
        r = X
    yg = jax.nn.relu(y + r)

    z4 = jnp.zeros((4 * G, cout), jnp.float32)
    sp[:4 * G] = z4
    sp[4 * G:4 * G + R] = yg
    sp[4 * G + R:] = z4
    acc = None
    for k in range(9):
        m = _mm(sp[k * G:k * G + R], p['tw'][k])
        acc = m if acc is None else acc + m
    acc = p['tg'] * (acc + p['tb']) + p['tbb']

    if last:
        out = jax.nn.relu(acc + X)
        sp[:R] = out
        tsum = jnp.sum(sp.reshape(T + 8, G, cout)[:T], axis=0)
        rmask = jax.lax.broadcasted_iota(jnp.int32, (_VP, 1), 0) < _V
        feats = jnp.concatenate(
            [jnp.sum(jnp.where(rmask, tsum[s * _VP:(s + 1) * _VP], 0.0),
                     axis=0, keepdims=True) / float(T * _V)
             for s in range(_S)], axis=0)
        o_ref[0] = _mm(feats, p['fc_w'])
        return
    if stride == 1:
        out = jax.nn.relu(acc + X) if residual else jax.nn.relu(acc)
        o_ref[0] = out
        return
    sp[:R] = acc
    acc3 = sp.reshape((T + 8) // 2, 2 * G, cout)[:T // 2, :G, :]
    rr = _mm(X, p['rw']) + p['rb']
    rr = p['rg'] * rr + p['rbb']
    sp[:R] = rr
    res3 = sp.reshape((T + 8) // 2, 2 * G, cout)[:T // 2, :G, :]
    out3 = jax.nn.relu(acc3 + res3)
    o_ref.reshape(T // 2, G, cout)[...] = out3


def _full(shape):
    nd = len(shape)
    return pl.BlockSpec(shape, lambda i: (0,) * nd)


def _scratch(cin, cout, T):
    G = _S * _VP
    R = T * G
    inter = cout // 4
    return [
        pltpu.VMEM((R, inter), jnp.float32),
        pltpu.VMEM((R, inter), jnp.float32),
        pltpu.VMEM((R, cin), jnp.float32),
        pltpu.VMEM(((T + 8) * G, cout), jnp.float32),
    ]


def kernel(x, params):
    N, C, T0, V, M = x.shape
    NM = N * M
    NB = NM // _S
    G = _S * _VP
    preps = [_prep_layer(lp) for lp in params['layers']]
    gpad = ((0, 0), (0, _VP - _V), (0, 0))
    reps = (_S // M, 1, 1)
    g_svc = jnp.tile(jnp.pad(params['data_bn']['g'].reshape(M, V, C), gpad), reps)
    b_svc = jnp.tile(jnp.pad(params['data_bn']['b'].reshape(M, V, C), gpad), reps)
    preps[9]['fc_w'] = params['fc_w']

    xr = jnp.pad(jnp.transpose(x, (0, 4, 2, 3, 1)),
                 ((0, 0), (0, 0), (0, 0), (0, _VP - _V), (0, 0)))
    xr = xr.reshape(NB, _S, T0, _VP, C).transpose(0, 2, 1, 3, 4)
    xr = xr.reshape(NB, T0 * G, C)
    cin, cout, stride, residual = _CFG[0]
    flat, treedef = jax.tree_util.tree_flatten(preps[0])

    def body1(x_ref, g_ref, b_ref, *rest):
        wrefs, o_ref = rest[:len(flat)], rest[len(flat)]
        sx = rest[len(flat) + 1]
        scr = rest[len(flat) + 2:]
        p = jax.tree_util.tree_unflatten(treedef, [r[...] for r in wrefs])
        x4in = x_ref.reshape(T0, _S, _VP, C)[...]
        gv, bv = g_ref[...], b_ref[...]
        sx.reshape(T0, _S, _VP, C)[...] = x4in * gv[None] + bv[None]
        _layer_body(sx[...], sx.reshape(T0, _S, _VP, C), p, o_ref, *scr,
                    cin, cout, T0, stride, residual, False)

    y = pl.pallas_call(
        body1,
        grid=(NB,),
        in_specs=[pl.BlockSpec((1, T0 * G, C), lambda i: (i, 0, 0)),
                  pl.BlockSpec((_S, _VP, C), lambda i: (0, 0, 0)),
                  pl.BlockSpec((_S, _VP, C), lambda i: (0, 0, 0))]
                 + [_full(f.shape) for f in flat],
        out_specs=pl.BlockSpec((1, T0 * G, cout), lambda i: (i, 0, 0)),
        out_shape=jax.ShapeDtypeStruct((NB, T0 * G, cout), jnp.float32),
        scratch_shapes=[pltpu.VMEM((T0 * G, C), jnp.float32)]
                       + _scratch(cin, cout, T0),
    )(xr, g_svc, b_svc, *flat)

    T = T0
    for li in range(1, 10):
        cin, cout, stride, residual = _CFG[li]
        last = li == 9
        flat, treedef = jax.tree_util.tree_flatten(preps[li])
        R = T * G
        Ro = (T // stride) * G

        def body(x_ref, *rest, _treedef=treedef, _n=len(flat), _cin=cin,
                 _cout=cout, _T=T, _stride=stride, _residual=residual, _last=last):
            wrefs, o_ref, scr = rest[:_n], rest[_n], rest[_n + 1:]
            p = jax.tree_util.tree_unflatten(_treedef, [r[...] for r in wrefs])
            _layer_body(x_ref[0], x_ref.reshape(_T, _S, _VP, _cin), p, o_ref,
                        *scr, _cin, _cout, _T, _stride, _residual, _last)

        if last:
            out_specs = pl.BlockSpec((1, _S, _NC), lambda i: (i, 0, 0))
            out_shape = jax.ShapeDtypeStruct((NB, _S, _NC), jnp.float32)
        else:
            out_specs = pl.BlockSpec((1, Ro, cout), lambda i: (i, 0, 0))
            out_shape = jax.ShapeDtypeStruct((NB, Ro, cout), jnp.float32)

        y = pl.pallas_call(
            body,
            grid=(NB,),
            in_specs=[pl.BlockSpec((1, R, cin), lambda i: (i, 0, 0))]
                     + [_full(f.shape) for f in flat],
            out_specs=out_specs,
            out_shape=out_shape,
            scratch_shapes=_scratch(cin, cout, T),
        )(y, *flat)
        T = T // stride

    return y.reshape(N, M, _NC).mean(axis=1) + params['fc_b'][None, :]


# speedup vs baseline: 1.0989x; 1.0989x over previous
import jax
import jax.numpy as jnp
from jax.experimental import pallas as pl
from jax.experimental.pallas import tpu as pltpu

_V = 27
_VP = 32
_S = 4
_CFG = [(3, 64, 1, False), (64, 64, 1, True), (64, 64, 1, True), (64, 64, 1, True),
        (64, 128, 2, True), (128, 128, 1, True), (128, 128, 1, True),
        (128, 256, 2, True), (256, 256, 1, True), (256, 256, 1, True)]

_NC = 60


def _r(v):
    return v.reshape(1, -1)


def _prep_layer(lp):
    g = lp['gcn']
    pa = jnp.pad(g['PA'], ((0, 0), (0, _VP - _V), (0, _VP - _V)))
    p = {
        'PA': pa,
        'wa': list(g['wa']), 'ba': [_r(b) for b in g['ba']],
        'wb': list(g['wb']), 'bb': [_r(b) for b in g['bb']],
        'wd': list(g['wd']), 'bd': [_r(b) for b in g['bd']],
        'bn_g': _r(g['bn_g']), 'bn_b': _r(g['bn_b']),
    }
    if 'down_w' in g:
        p['down_w'] = g['down_w']
        p['down_b'] = _r(g['down_b'])
        p['down_g'] = _r(g['down_g'])
        p['down_bb'] = _r(g['down_bb'])
    t = lp['tcn']
    p['tw'] = jnp.transpose(t['w'][:, :, :, 0], (2, 1, 0))
    p['tb'] = _r(t['b'])
    p['tg'] = _r(t['g'])
    p['tbb'] = _r(t['bb'])
    if 'res' in lp:
        r = lp['res']
        p['rw'] = r['w'][:, :, 0, 0].T
        p['rb'] = _r(r['b'])
        p['rg'] = _r(r['g'])
        p['rbb'] = _r(r['bb'])
    return p


def _mm(a, b):
    return jnp.dot(a, b, preferred_element_type=jnp.float32)


def _dg(a, b, dims):
    return jax.lax.dot_general(a, b, dims, preferred_element_type=jnp.float32)


def _layer_body(X, x3get, p, o_ref, sa, sb, sxa, sp,
                cin, cout, T, stride, residual, last):
    TV = T * _VP
    inter = cout // 4
    x3 = x3get()
    X_wide = jnp.concatenate([x3[t] for t in range(T)], axis=1)
    sa3 = sa.reshape(T, _VP, inter)
    sb3 = sb.reshape(T, _VP, inter)
    sxa3 = sxa.reshape(T, _VP, cin)
    vmask = jax.lax.broadcasted_iota(jnp.int32, (_VP, _VP), 0) < _V
    y = None
    for i in range(3):
        sa[...] = _mm(X, p['wa'][i]) + p['ba'][i]
        sb[...] = _mm(X, p['wb'][i]) + p['bb'][i]
        c1 = jnp.concatenate([sa3[t] for t in range(T)], axis=1)
        c2 = jnp.concatenate([sb3[t] for t in range(T)], axis=1)
        logits = _dg(c1, c2, (((1,), (1,)), ((), ()))) / float(inter * T)
        logits = jnp.where(vmask, logits, -1e30)
        mx = jnp.max(logits, axis=0, keepdims=True)
        e = jnp.exp(logits - mx)
        att = e / jnp.sum(e, axis=0, keepdims=True)
        A1 = att + p['PA'][i]
        xa_wide = _dg(A1, X_wide, (((0,), (0,)), ((), ())))
        for t in range(T):
            sxa3[t] = xa_wide[:, t * cin:(t + 1) * cin]
        contrib = _mm(sxa[...], p['wd'][i]) + p['bd'][i]
        y = contrib if y is None else y + contrib
    y = p['bn_g'] * y + p['bn_b']
    if 'down_w' in p:
        r = _mm(X, p['down_w']) + p['down_b']
        r = p['down_g'] * r + p['down_bb']
    else:
        r = X
    yg = jax.nn.relu(y + r)

    z4 = jnp.zeros((4 * _VP, cout), jnp.float32)
    sp[:4 * _VP] = z4
    sp[4 * _VP:4 * _VP + TV] = yg
    sp[4 * _VP + TV:] = z4
    acc = None
    for k in range(9):
        m = _mm(sp[k * _VP:k * _VP + TV], p['tw'][k])
        acc = m if acc is None else acc + m
    acc = p['tg'] * (acc + p['tb']) + p['tbb']

    if last:
        out = jax.nn.relu(acc + X)
        rmask = jax.lax.broadcasted_iota(jnp.int32, (TV, 1), 0) % _VP < _V
        masked = jnp.where(rmask, out, 0.0)
        feat = jnp.sum(masked, axis=0, keepdims=True) / float(T * _V)
        o_ref[...] = _mm(feat, p['fc_w'])
        return
    if stride == 1:
        out = jax.nn.relu(acc + X) if residual else jax.nn.relu(acc)
        o_ref[...] = out
        return
    sp[:TV] = acc
    acc3 = sp.reshape((T + 8) // 2, 2 * _VP, cout)[:T // 2, :_VP, :]
    rr = _mm(X, p['rw']) + p['rb']
    rr = p['rg'] * rr + p['rbb']
    sp[:TV] = rr
    res3 = sp.reshape((T + 8) // 2, 2 * _VP, cout)[:T // 2, :_VP, :]
    out3 = jax.nn.relu(acc3 + res3)
    o_ref.reshape(T // 2, _VP, cout)[...] = out3


def _viewer(ref, T, VP, C):
    return lambda: ref.reshape(T, VP, C)[...]


def _sviewer(ref, s, S, T, C):
    return lambda: ref.reshape(S, T, _VP, C)[s]


def _full(shape):
    nd = len(shape)
    return pl.BlockSpec(shape, lambda i: (0,) * nd)


def _scratch(cin, cout, T):
    TV = T * _VP
    inter = cout // 4
    return [
        pltpu.VMEM((TV, inter), jnp.float32),
        pltpu.VMEM((TV, inter), jnp.float32),
        pltpu.VMEM((TV, cin), jnp.float32),
        pltpu.VMEM(((T + 8) * _VP, cout), jnp.float32),
    ]


def kernel(x, params):
    N, C, T0, V, M = x.shape
    NM = N * M
    preps = [_prep_layer(lp) for lp in params['layers']]
    gpad = ((0, 0), (0, _VP - _V), (0, 0))
    g_mvc = jnp.pad(params['data_bn']['g'].reshape(M, V, C), gpad)
    b_mvc = jnp.pad(params['data_bn']['b'].reshape(M, V, C), gpad)
    preps[9]['fc_w'] = params['fc_w']

    S = _S
    xr = jnp.pad(jnp.transpose(x, (0, 4, 2, 3, 1)),
                 ((0, 0), (0, 0), (0, 0), (0, _VP - _V), (0, 0)))
    xr = xr.reshape(NM, T0 * _VP, C)
    cin, cout, stride, residual = _CFG[0]
    flat, treedef = jax.tree_util.tree_flatten(preps[0])
    TV = T0 * _VP

    def body1(x_ref, g_ref, b_ref, *rest):
        wrefs, o_ref = rest[:len(flat)], rest[len(flat)]
        scr = rest[len(flat) + 1:]
        p = jax.tree_util.tree_unflatten(treedef, [r[...] for r in wrefs])
        for s in range(S):
            sx = scr[5 * s]
            x3in = x_ref.reshape(S, T0, _VP, C)[s]
            sx.reshape(T0, _VP, C)[...] = (x3in * g_ref[s % M][None]
                                           + b_ref[s % M][None])
            _layer_body(sx[...], _viewer(sx, T0, _VP, C), p, o_ref.at[s],
                        *scr[5 * s + 1:5 * s + 5],
                        cin, cout, T0, stride, residual, False)

    y = pl.pallas_call(
        body1,
        grid=(NM // S,),
        in_specs=[pl.BlockSpec((S, TV, C), lambda i: (i, 0, 0)),
                  pl.BlockSpec((M, _VP, C), lambda i: (0, 0, 0)),
                  pl.BlockSpec((M, _VP, C), lambda i: (0, 0, 0))]
                 + [_full(f.shape) for f in flat],
        out_specs=pl.BlockSpec((S, TV, cout), lambda i: (i, 0, 0)),
        out_shape=jax.ShapeDtypeStruct((NM, TV, cout), jnp.float32),
        scratch_shapes=([pltpu.VMEM((TV, C), jnp.float32)]
                        + _scratch(cin, cout, T0)) * S,
    )(xr, g_mvc, b_mvc, *flat)

    T = T0
    for li in range(1, 10):
        cin, cout, stride, residual = _CFG[li]
        last = li == 9
        flat, treedef = jax.tree_util.tree_flatten(preps[li])
        TV = T * _VP
        TVo = (T // stride) * _VP

        def body(x_ref, *rest, _treedef=treedef, _n=len(flat), _cin=cin,
                 _cout=cout, _T=T, _stride=stride, _residual=residual, _last=last):
            wrefs, o_ref, scr = rest[:_n], rest[_n], rest[_n + 1:]
            p = jax.tree_util.tree_unflatten(_treedef, [r[...] for r in wrefs])
            for s in range(S):
                _layer_body(x_ref[s], _sviewer(x_ref, s, S, _T, _cin), p,
                            o_ref.at[s], *scr[4 * s:4 * s + 4],
                            _cin, _cout, _T, _stride, _residual, _last)

        if last:
            out_specs = pl.BlockSpec((S, 1, _NC), lambda i: (i, 0, 0))
            out_shape = jax.ShapeDtypeStruct((NM, 1, _NC), jnp.float32)
        else:
            out_specs = pl.BlockSpec((S, TVo, cout), lambda i: (i, 0, 0))
            out_shape = jax.ShapeDtypeStruct((NM, TVo, cout), jnp.float32)

        y = pl.pallas_call(
            body,
            grid=(NM // S,),
            in_specs=[pl.BlockSpec((S, TV, cin), lambda i: (i, 0, 0))]
                     + [_full(f.shape) for f in flat],
            out_specs=out_specs,
            out_shape=out_shape,
            scratch_shapes=_scratch(cin, cout, T) * S,
        )(y, *flat)
        T = T // stride

    return y.reshape(N, M, _NC).mean(axis=1) + params['fc_b'][None, :]


# speedup vs baseline: 1.1012x; 1.0022x over previous
import jax
import jax.numpy as jnp
from jax.experimental import pallas as pl
from jax.experimental.pallas import tpu as pltpu

_V = 27
_VP = 32
_S = 4
_CFG = [(3, 64, 1, False), (64, 64, 1, True), (64, 64, 1, True), (64, 64, 1, True),
        (64, 128, 2, True), (128, 128, 1, True), (128, 128, 1, True),
        (128, 256, 2, True), (256, 256, 1, True), (256, 256, 1, True)]

_NC = 60


def _r(v):
    return v.reshape(1, -1)


def _prep_layer(lp):
    g = lp['gcn']
    pa = jnp.pad(g['PA'], ((0, 0), (0, _VP - _V), (0, _VP - _V)))
    p = {
        'PA': pa,
        'wa': [w.astype(jnp.bfloat16) for w in g['wa']], 'ba': [_r(b) for b in g['ba']],
        'wb': [w.astype(jnp.bfloat16) for w in g['wb']], 'bb': [_r(b) for b in g['bb']],
        'wd': [w.astype(jnp.bfloat16) for w in g['wd']], 'bd': [_r(b) for b in g['bd']],
        'bn_g': _r(g['bn_g']), 'bn_b': _r(g['bn_b']),
    }
    if 'down_w' in g:
        p['down_w'] = g['down_w'].astype(jnp.bfloat16)
        p['down_b'] = _r(g['down_b'])
        p['down_g'] = _r(g['down_g'])
        p['down_bb'] = _r(g['down_bb'])
    t = lp['tcn']
    p['tw'] = jnp.transpose(t['w'][:, :, :, 0], (2, 1, 0)).astype(jnp.bfloat16)
    p['tb'] = _r(t['b'])
    p['tg'] = _r(t['g'])
    p['tbb'] = _r(t['bb'])
    if 'res' in lp:
        r = lp['res']
        p['rw'] = r['w'][:, :, 0, 0].T.astype(jnp.bfloat16)
        p['rb'] = _r(r['b'])
        p['rg'] = _r(r['g'])
        p['rbb'] = _r(r['bb'])
    return p


def _mm(a, b):
    return jnp.dot(a.astype(jnp.bfloat16), b, preferred_element_type=jnp.float32)


def _dg(a, b, dims):
    return jax.lax.dot_general(a.astype(jnp.bfloat16), b.astype(jnp.bfloat16),
                               dims, preferred_element_type=jnp.float32)


def _layer_body(X, x3get, p, o_ref, sa, sb, sxa, sp,
                cin, cout, T, stride, residual, last):
    TV = T * _VP
    inter = cout // 4
    x3 = x3get()
    X_wide = jnp.concatenate([x3[t] for t in range(T)], axis=1)
    sa3 = sa.reshape(T, _VP, inter)
    sb3 = sb.reshape(T, _VP, inter)
    sxa3 = sxa.reshape(T, _VP, cin)
    vmask = jax.lax.broadcasted_iota(jnp.int32, (_VP, _VP), 0) < _V
    y = None
    for i in range(3):
        sa[...] = _mm(X, p['wa'][i]) + p['ba'][i]
        sb[...] = _mm(X, p['wb'][i]) + p['bb'][i]
        c1 = jnp.concatenate([sa3[t] for t in range(T)], axis=1)
        c2 = jnp.concatenate([sb3[t] for t in range(T)], axis=1)
        logits = _dg(c1, c2, (((1,), (1,)), ((), ()))) / float(inter * T)
        logits = jnp.where(vmask, logits, -1e30)
        mx = jnp.max(logits, axis=0, keepdims=True)
        e = jnp.exp(logits - mx)
        att = e / jnp.sum(e, axis=0, keepdims=True)
        A1 = att + p['PA'][i]
        xa_wide = _dg(A1, X_wide, (((0,), (0,)), ((), ())))
        for t in range(T):
            sxa3[t] = xa_wide[:, t * cin:(t + 1) * cin]
        contrib = _mm(sxa[...], p['wd'][i]) + p['bd'][i]
        y = contrib if y is None else y + contrib
    y = p['bn_g'] * y + p['bn_b']
    if 'down_w' in p:
        r = _mm(X, p['down_w']) + p['down_b']
        r = p['down_g'] * r + p['down_bb']
    else:
        r = X
    yg = jax.nn.relu(y + r)

    z4 = jnp.zeros((4 * _VP, cout), jnp.float32)
    sp[:4 * _VP] = z4
    sp[4 * _VP:4 * _VP + TV] = yg
    sp[4 * _VP + TV:] = z4
    acc = None
    for k in range(9):
        m = _mm(sp[k * _VP:k * _VP + TV], p['tw'][k])
        acc = m if acc is None else acc + m
    acc = p['tg'] * (acc + p['tb']) + p['tbb']

    if last:
        out = jax.nn.relu(acc + X)
        rmask = jax.lax.broadcasted_iota(jnp.int32, (TV, 1), 0) % _VP < _V
        masked = jnp.where(rmask, out, 0.0)
        feat = jnp.sum(masked, axis=0, keepdims=True) / float(T * _V)
        o_ref[...] = _mm(feat, p['fc_w'])
        return
    if stride == 1:
        out = jax.nn.relu(acc + X) if residual else jax.nn.relu(acc)
        o_ref[...] = out
        return
    sp[:TV] = acc
    acc3 = sp.reshape((T + 8) // 2, 2 * _VP, cout)[:T // 2, :_VP, :]
    rr = _mm(X, p['rw']) + p['rb']
    rr = p['rg'] * rr + p['rbb']
    sp[:TV] = rr
    res3 = sp.reshape((T + 8) // 2, 2 * _VP, cout)[:T // 2, :_VP, :]
    out3 = jax.nn.relu(acc3 + res3)
    o_ref.reshape(T // 2, _VP, cout)[...] = out3


def _viewer(ref, T, VP, C):
    return lambda: ref.reshape(T, VP, C)[...]


def _sviewer(ref, s, S, T, C):
    return lambda: ref.reshape(S, T, _VP, C)[s]


def _full(shape):
    nd = len(shape)
    return pl.BlockSpec(shape, lambda i: (0,) * nd)


def _scratch(cin, cout, T):
    TV = T * _VP
    inter = cout // 4
    return [
        pltpu.VMEM((TV, inter), jnp.float32),
        pltpu.VMEM((TV, inter), jnp.float32),
        pltpu.VMEM((TV, cin), jnp.float32),
        pltpu.VMEM(((T + 8) * _VP, cout), jnp.float32),
    ]


def kernel(x, params):
    N, C, T0, V, M = x.shape
    NM = N * M
    preps = [_prep_layer(lp) for lp in params['layers']]
    gpad = ((0, 0), (0, _VP - _V), (0, 0))
    g_mvc = jnp.pad(params['data_bn']['g'].reshape(M, V, C), gpad)
    b_mvc = jnp.pad(params['data_bn']['b'].reshape(M, V, C), gpad)
    preps[9]['fc_w'] = params['fc_w'].astype(jnp.bfloat16)

    S = _S
    xr = jnp.pad(jnp.transpose(x, (0, 4, 2, 3, 1)),
                 ((0, 0), (0, 0), (0, 0), (0, _VP - _V), (0, 0)))
    xr = xr.reshape(NM, T0 * _VP, C)
    cin, cout, stride, residual = _CFG[0]
    flat, treedef = jax.tree_util.tree_flatten(preps[0])
    TV = T0 * _VP

    def body1(x_ref, g_ref, b_ref, *rest):
        wrefs, o_ref = rest[:len(flat)], rest[len(flat)]
        scr = rest[len(flat) + 1:]
        p = jax.tree_util.tree_unflatten(treedef, [r[...] for r in wrefs])
        for s in range(S):
            sx = scr[5 * s]
            x3in = x_ref.reshape(S, T0, _VP, C)[s]
            sx.reshape(T0, _VP, C)[...] = (x3in * g_ref[s % M][None]
                                           + b_ref[s % M][None])
            _layer_body(sx[...], _viewer(sx, T0, _VP, C), p, o_ref.at[s],
                        *scr[5 * s + 1:5 * s + 5],
                        cin, cout, T0, stride, residual, False)

    y = pl.pallas_call(
        body1,
        grid=(NM // S,),
        in_specs=[pl.BlockSpec((S, TV, C), lambda i: (i, 0, 0)),
                  pl.BlockSpec((M, _VP, C), lambda i: (0, 0, 0)),
                  pl.BlockSpec((M, _VP, C), lambda i: (0, 0, 0))]
                 + [_full(f.shape) for f in flat],
        out_specs=pl.BlockSpec((S, TV, cout), lambda i: (i, 0, 0)),
        out_shape=jax.ShapeDtypeStruct((NM, TV, cout), jnp.float32),
        scratch_shapes=([pltpu.VMEM((TV, C), jnp.float32)]
                        + _scratch(cin, cout, T0)) * S,
    )(xr, g_mvc, b_mvc, *flat)

    T = T0
    for li in range(1, 10):
        cin, cout, stride, residual = _CFG[li]
        last = li == 9
        flat, treedef = jax.tree_util.tree_flatten(preps[li])
        TV = T * _VP
        TVo = (T // stride) * _VP

        def body(x_ref, *rest, _treedef=treedef, _n=len(flat), _cin=cin,
                 _cout=cout, _T=T, _stride=stride, _residual=residual, _last=last):
            wrefs, o_ref, scr = rest[:_n], rest[_n], rest[_n + 1:]
            p = jax.tree_util.tree_unflatten(_treedef, [r[...] for r in wrefs])
            for s in range(S):
                _layer_body(x_ref[s], _sviewer(x_ref, s, S, _T, _cin), p,
                            o_ref.at[s], *scr[4 * s:4 * s + 4],
                            _cin, _cout, _T, _stride, _residual, _last)

        if last:
            out_specs = pl.BlockSpec((S, 1, _NC), lambda i: (i, 0, 0))
            out_shape = jax.ShapeDtypeStruct((NM, 1, _NC), jnp.float32)
        else:
            out_specs = pl.BlockSpec((S, TVo, cout), lambda i: (i, 0, 0))
            out_shape = jax.ShapeDtypeStruct((NM, TVo, cout), jnp.float32)

        y = pl.pallas_call(
            body,
            grid=(NM // S,),
            in_specs=[pl.BlockSpec((S, TV, cin), lambda i: (i, 0, 0))]
                     + [_full(f.shape) for f in flat],
            out_specs=out_specs,
            out_shape=out_shape,
            scratch_shapes=_scratch(cin, cout, T) * S,
        )(y, *flat)
        T = T // stride

    return y.reshape(N, M, _NC).mean(axis=1) + params['fc_b'][None, :]


# speedup vs baseline: 1.3161x; 1.1951x over previous
import jax
import jax.numpy as jnp
from jax.experimental import pallas as pl
from jax.experimental.pallas import tpu as pltpu

_V = 27
_VP = 32
_S = 4
_CFG = [(3, 64, 1, False), (64, 64, 1, True), (64, 64, 1, True), (64, 64, 1, True),
        (64, 128, 2, True), (128, 128, 1, True), (128, 128, 1, True),
        (128, 256, 2, True), (256, 256, 1, True), (256, 256, 1, True)]

_NC = 60


def _r(v):
    return v.reshape(1, -1)


def _prep_layer(lp):
    g = lp['gcn']
    pa = jnp.pad(g['PA'], ((0, 0), (0, _VP - _V), (0, _VP - _V)))
    p = {
        'PA': pa,
        'wab': [jnp.concatenate([wa, wb], axis=1).astype(jnp.bfloat16)
                for wa, wb in zip(g['wa'], g['wb'])],
        'bab': [jnp.concatenate([ba, bb]).reshape(1, -1)
                for ba, bb in zip(g['ba'], g['bb'])],
        'wd': jnp.concatenate(g['wd'], axis=0).astype(jnp.bfloat16),
        'bd': _r(g['bd'][0] + g['bd'][1] + g['bd'][2]),
        'bn_g': _r(g['bn_g']), 'bn_b': _r(g['bn_b']),
    }
    if 'down_w' in g:
        p['down_w'] = g['down_w'].astype(jnp.bfloat16)
        p['down_b'] = _r(g['down_b'])
        p['down_g'] = _r(g['down_g'])
        p['down_bb'] = _r(g['down_bb'])
    t = lp['tcn']
    p['tw'] = jnp.transpose(t['w'][:, :, :, 0], (2, 1, 0)).astype(jnp.bfloat16)
    p['tb'] = _r(t['b'])
    p['tg'] = _r(t['g'])
    p['tbb'] = _r(t['bb'])
    if 'res' in lp:
        r = lp['res']
        p['rw'] = r['w'][:, :, 0, 0].T.astype(jnp.bfloat16)
        p['rb'] = _r(r['b'])
        p['rg'] = _r(r['g'])
        p['rbb'] = _r(r['bb'])
    return p


def _mm(a, b):
    return jnp.dot(a.astype(jnp.bfloat16), b, preferred_element_type=jnp.float32)


def _dg(a, b, dims):
    return jax.lax.dot_general(a.astype(jnp.bfloat16), b.astype(jnp.bfloat16),
                               dims, preferred_element_type=jnp.float32)


def _layer_body(X, x3get, p, o_ref, sa, sxa, sp,
                cin, cout, T, stride, residual, last):
    TV = T * _VP
    inter = cout // 4
    x3 = x3get()
    X_wide = jnp.concatenate([x3[t] for t in range(T)], axis=1)
    sa3 = sa.reshape(T, _VP, 2 * inter)
    sxa3 = sxa.reshape(T, _VP, 3 * cin)
    vmask = jax.lax.broadcasted_iota(jnp.int32, (_VP, _VP), 0) < _V
    y = None
    for i in range(3):
        sa[...] = _mm(X, p['wab'][i]) + p['bab'][i]
        c1 = jnp.concatenate([sa3[t][:, :inter] for t in range(T)], axis=1)
        c2 = jnp.concatenate([sa3[t][:, inter:] for t in range(T)], axis=1)
        logits = _dg(c1, c2, (((1,), (1,)), ((), ()))) / float(inter * T)
        logits = jnp.where(vmask, logits, -1e30)
        mx = jnp.max(logits, axis=0, keepdims=True)
        e = jnp.exp(logits - mx)
        att = e / jnp.sum(e, axis=0, keepdims=True)
        A1 = att + p['PA'][i]
        xa_wide = _dg(A1, X_wide, (((0,), (0,)), ((), ())))
        for t in range(T):
            sxa3[t, :, i * cin:(i + 1) * cin] = xa_wide[:, t * cin:(t + 1) * cin]
    y = _mm(sxa[...], p['wd']) + p['bd']
    y = p['bn_g'] * y + p['bn_b']
    if 'down_w' in p:
        r = _mm(X, p['down_w']) + p['down_b']
        r = p['down_g'] * r + p['down_bb']
    else:
        r = X
    yg = jax.nn.relu(y + r)

    z4 = jnp.zeros((4 * _VP, cout), jnp.float32)
    sp[:4 * _VP] = z4
    sp[4 * _VP:4 * _VP + TV] = yg
    sp[4 * _VP + TV:] = z4
    acc = None
    for k in range(9):
        m = _mm(sp[k * _VP:k * _VP + TV], p['tw'][k])
        acc = m if acc is None else acc + m
    acc = p['tg'] * (acc + p['tb']) + p['tbb']

    if last:
        out = jax.nn.relu(acc + X)
        rmask = jax.lax.broadcasted_iota(jnp.int32, (TV, 1), 0) % _VP < _V
        masked = jnp.where(rmask, out, 0.0)
        feat = jnp.sum(masked, axis=0, keepdims=True) / float(T * _V)
        o_ref[...] = _mm(feat, p['fc_w'])
        return
    if stride == 1:
        out = jax.nn.relu(acc + X) if residual else jax.nn.relu(acc)
        o_ref[...] = out
        return
    sp[:TV] = acc
    acc3 = sp.reshape((T + 8) // 2, 2 * _VP, cout)[:T // 2, :_VP, :]
    rr = _mm(X, p['rw']) + p['rb']
    rr = p['rg'] * rr + p['rbb']
    sp[:TV] = rr
    res3 = sp.reshape((T + 8) // 2, 2 * _VP, cout)[:T // 2, :_VP, :]
    out3 = jax.nn.relu(acc3 + res3)
    o_ref.reshape(T // 2, _VP, cout)[...] = out3


def _viewer(ref, T, VP, C):
    return lambda: ref.reshape(T, VP, C)[...]


def _sviewer(ref, s, S, T, C):
    return lambda: ref.reshape(S, T, _VP, C)[s]


def _full(shape):
    nd = len(shape)
    return pl.BlockSpec(shape, lambda i: (0,) * nd)


def _scratch(cin, cout, T):
    TV = T * _VP
    inter = cout // 4
    return [
        pltpu.VMEM((TV, 2 * inter), jnp.float32),
        pltpu.VMEM((TV, 3 * cin), jnp.float32),
        pltpu.VMEM(((T + 8) * _VP, cout), jnp.float32),
    ]


def kernel(x, params):
    N, C, T0, V, M = x.shape
    NM = N * M
    preps = [_prep_layer(lp) for lp in params['layers']]
    gpad = ((0, 0), (0, _VP - _V), (0, 0))
    g_mvc = jnp.pad(params['data_bn']['g'].reshape(M, V, C), gpad)
    b_mvc = jnp.pad(params['data_bn']['b'].reshape(M, V, C), gpad)
    preps[9]['fc_w'] = params['fc_w'].astype(jnp.bfloat16)

    S = _S
    xr = jnp.pad(jnp.transpose(x, (0, 4, 2, 3, 1)),
                 ((0, 0), (0, 0), (0, 0), (0, _VP - _V), (0, 0)))
    xr = xr.reshape(NM, T0 * _VP, C)
    cin, cout, stride, residual = _CFG[0]
    flat, treedef = jax.tree_util.tree_flatten(preps[0])
    TV = T0 * _VP

    def body1(x_ref, g_ref, b_ref, *rest):
        wrefs, o_ref = rest[:len(flat)], rest[len(flat)]
        scr = rest[len(flat) + 1:]
        p = jax.tree_util.tree_unflatten(treedef, [r[...] for r in wrefs])
        for s in range(S):
            sx = scr[4 * s]
            x3in = x_ref.reshape(S, T0, _VP, C)[s]
            sx.reshape(T0, _VP, C)[...] = (x3in * g_ref[s % M][None]
                                           + b_ref[s % M][None])
            _layer_body(sx[...], _viewer(sx, T0, _VP, C), p, o_ref.at[s],
                        *scr[4 * s + 1:4 * s + 4],
                        cin, cout, T0, stride, residual, False)

    y = pl.pallas_call(
        body1,
        grid=(NM // S,),
        in_specs=[pl.BlockSpec((S, TV, C), lambda i: (i, 0, 0)),
                  pl.BlockSpec((M, _VP, C), lambda i: (0, 0, 0)),
                  pl.BlockSpec((M, _VP, C), lambda i: (0, 0, 0))]
                 + [_full(f.shape) for f in flat],
        out_specs=pl.BlockSpec((S, TV, cout), lambda i: (i, 0, 0)),
        out_shape=jax.ShapeDtypeStruct((NM, TV, cout), jnp.float32),
        scratch_shapes=([pltpu.VMEM((TV, C), jnp.float32)]
                        + _scratch(cin, cout, T0)) * S,
    )(xr, g_mvc, b_mvc, *flat)

    T = T0
    for li in range(1, 10):
        cin, cout, stride, residual = _CFG[li]
        last = li == 9
        flat, treedef = jax.tree_util.tree_flatten(preps[li])
        TV = T * _VP
        TVo = (T // stride) * _VP

        def body(x_ref, *rest, _treedef=treedef, _n=len(flat), _cin=cin,
                 _cout=cout, _T=T, _stride=stride, _residual=residual, _last=last):
            wrefs, o_ref, scr = rest[:_n], rest[_n], rest[_n + 1:]
            p = jax.tree_util.tree_unflatten(_treedef, [r[...] for r in wrefs])
            for s in range(S):
                _layer_body(x_ref[s], _sviewer(x_ref, s, S, _T, _cin), p,
                            o_ref.at[s], *scr[3 * s:3 * s + 3],
                            _cin, _cout, _T, _stride, _residual, _last)

        if last:
            out_specs = pl.BlockSpec((S, 1, _NC), lambda i: (i, 0, 0))
            out_shape = jax.ShapeDtypeStruct((NM, 1, _NC), jnp.float32)
        else:
            out_specs = pl.BlockSpec((S, TVo, cout), lambda i: (i, 0, 0))
            out_shape = jax.ShapeDtypeStruct((NM, TVo, cout), jnp.float32)

        y = pl.pallas_call(
            body,
            grid=(NM // S,),
            in_specs=[pl.BlockSpec((S, TV, cin), lambda i: (i, 0, 0))]
                     + [_full(f.shape) for f in flat],
            out_specs=out_specs,
            out_shape=out_shape,
            scratch_shapes=_scratch(cin, cout, T) * S,
        )(y, *flat)
        T = T // stride

    return y.reshape(N, M, _NC).mean(axis=1) + params['fc_b'][None, :]
